# baseline (speedup 1.0000x reference)
import functools

import jax
import jax.numpy as jnp
from jax import lax
from jax.experimental import pallas as pl
from jax.experimental.pallas import tpu as pltpu

N_EMB = 1024
DIM = 64
T_TOK = 16384
BLK = 512
N_BLK = T_TOK // BLK
TBLK = BLK // 16
DECAY_C = 0.99
COMMIT_C = 0.25
EPS_C = 1e-05


def _pass1_body(inp_ref, emb_ref, dist_ref, enc_ref, idx_ref, cnt_ref, dw_ref):
    i = pl.program_id(0)
    x3 = inp_ref[...]
    flat = x3.reshape(DIM, BLK).T
    e = emb_ref[...]
    xs = jnp.sum(flat * flat, axis=1, keepdims=True)
    es = jnp.sum(e * e, axis=1)
    xe = lax.dot_general(flat, e, (((1,), (1,)), ((), ())))
    dist = xs + es[None, :] - 2.0 * xe
    dist_ref[...] = dist
    idx = jnp.argmin(dist, axis=1).astype(jnp.int32)
    idx_ref[...] = idx.reshape(1, 1, BLK)
    cols = lax.broadcasted_iota(jnp.int32, (BLK, N_EMB), 1)
    enc = (cols == idx[:, None]).astype(jnp.float32)
    enc_ref[...] = enc
    cnt = jnp.sum(enc, axis=0)
    dwp = lax.dot_general(enc, flat, (((0,), (0,)), ((), ())))

    @pl.when(i == 0)
    def _():
        cnt_ref[...] = cnt[None, :]
        dw_ref[...] = dwp

    @pl.when(i > 0)
    def _():
        cnt_ref[...] += cnt[None, :]
        dw_ref[...] += dwp


def _pass2_body(inp_ref, idx_ref, cnt_ref, dw_ref, emaw_ref, emacs_ref,
                q_ref, loss_ref, perp_ref, w_scr, acc_scr):
    i = pl.program_id(0)

    @pl.when(i == 0)
    def _():
        counts = cnt_ref[0, :]
        cs = emacs_ref[0, :] * DECAY_C + (1.0 - DECAY_C) * counts
        n = jnp.sum(cs)
        csn = (cs + EPS_C) / (n + N_EMB * EPS_C) * n
        w = (emaw_ref[...] * DECAY_C + (1.0 - DECAY_C) * dw_ref[...])
        w_scr[...] = w / csn[:, None]
        p = counts * (1.0 / T_TOK)
        perp_ref[...] = jnp.exp(-jnp.sum(p * jnp.log(p + 1e-10))).reshape(1, 1)
        acc_scr[...] = jnp.zeros((1, 1), jnp.float32)

    idx = idx_ref[0, 0, :]
    cols = lax.broadcasted_iota(jnp.int32, (BLK, N_EMB), 1)
    enc = (cols == idx[:, None]).astype(jnp.float32)
    q = lax.dot_general(enc, w_scr[...], (((1,), (0,)), ((), ())))
    x3 = inp_ref[...]
    flat = x3.reshape(DIM, BLK).T
    d = q - flat
    acc_scr[...] = acc_scr[...] + jnp.sum(d * d).reshape(1, 1)
    q_ref[...] = q.T.reshape(DIM, TBLK, 16)

    @pl.when(i == N_BLK - 1)
    def _():
        loss_ref[...] = acc_scr[...] * (COMMIT_C / (T_TOK * DIM))


@jax.jit
def kernel(inputs, embedding_weight, ema_w, ema_cluster_size):
    dist, enc, idx, cnt, dw = pl.pallas_call(
        _pass1_body,
        grid=(N_BLK,),
        in_specs=[
            pl.BlockSpec((DIM, TBLK, 16), lambda i: (0, i, 0)),
            pl.BlockSpec((N_EMB, DIM), lambda i: (0, 0)),
        ],
        out_specs=[
            pl.BlockSpec((BLK, N_EMB), lambda i: (i, 0)),
            pl.BlockSpec((BLK, N_EMB), lambda i: (i, 0)),
            pl.BlockSpec((1, 1, BLK), lambda i: (i, 0, 0)),
            pl.BlockSpec((1, N_EMB), lambda i: (0, 0)),
            pl.BlockSpec((N_EMB, DIM), lambda i: (0, 0)),
        ],
        out_shape=[
            jax.ShapeDtypeStruct((T_TOK, N_EMB), jnp.float32),
            jax.ShapeDtypeStruct((T_TOK, N_EMB), jnp.float32),
            jax.ShapeDtypeStruct((N_BLK, 1, BLK), jnp.int32),
            jax.ShapeDtypeStruct((1, N_EMB), jnp.float32),
            jax.ShapeDtypeStruct((N_EMB, DIM), jnp.float32),
        ],
    )(inputs, embedding_weight)

    q, loss, perp = pl.pallas_call(
        _pass2_body,
        grid=(N_BLK,),
        in_specs=[
            pl.BlockSpec((DIM, TBLK, 16), lambda i: (0, i, 0)),
            pl.BlockSpec((1, 1, BLK), lambda i: (i, 0, 0)),
            pl.BlockSpec((1, N_EMB), lambda i: (0, 0)),
            pl.BlockSpec((N_EMB, DIM), lambda i: (0, 0)),
            pl.BlockSpec((N_EMB, DIM), lambda i: (0, 0)),
            pl.BlockSpec((1, N_EMB), lambda i: (0, 0)),
        ],
        out_specs=[
            pl.BlockSpec((DIM, TBLK, 16), lambda i: (0, i, 0)),
            pl.BlockSpec((1, 1), lambda i: (0, 0)),
            pl.BlockSpec((1, 1), lambda i: (0, 0)),
        ],
        out_shape=[
            jax.ShapeDtypeStruct((DIM, 1024, 16), jnp.float32),
            jax.ShapeDtypeStruct((1, 1), jnp.float32),
            jax.ShapeDtypeStruct((1, 1), jnp.float32),
        ],
        scratch_shapes=[
            pltpu.VMEM((N_EMB, DIM), jnp.float32),
            pltpu.VMEM((1, 1), jnp.float32),
        ],
    )(inputs, idx, cnt, dw, ema_w, ema_cluster_size.reshape(1, N_EMB))

    return (loss[0, 0], q, perp[0, 0], enc, dist)
